# baseline (speedup 1.0000x reference)
import functools

import jax
import jax.numpy as jnp
from jax import lax
from jax.experimental import pallas as pl
from jax.experimental.pallas import tpu as pltpu
from jax.experimental.pallas import tpu_sc as plsc

BATCH = 16384
EMBED_DIM = 16
CHUNK = 128


@jax.jit
def _gmf(user, item, user_table, item_table):
    info = plsc.get_sparse_core_info()
    nc, ns = info.num_cores, info.num_subcores
    nw = nc * ns
    b_per_w = BATCH // nw
    n_chunks = b_per_w // CHUNK

    user2 = user.reshape(nw, n_chunks, CHUNK).astype(jnp.int32)
    item2 = item.reshape(nw, n_chunks, CHUNK).astype(jnp.int32)

    mesh = plsc.VectorSubcoreMesh(core_axis_name="c", subcore_axis_name="s")

    @functools.partial(
        pl.kernel,
        out_type=jax.ShapeDtypeStruct((nw, n_chunks, CHUNK, EMBED_DIM), jnp.float32),
        mesh=mesh,
        compiler_params=pltpu.CompilerParams(use_tc_tiling_on_sc=False),
        scratch_types=[
            pltpu.VMEM((n_chunks, CHUNK), jnp.int32),
            pltpu.VMEM((n_chunks, CHUNK), jnp.int32),
            pltpu.VMEM((n_chunks, CHUNK, EMBED_DIM), jnp.float32),
            pltpu.VMEM((n_chunks, CHUNK, EMBED_DIM), jnp.float32),
            pltpu.SemaphoreType.DMA,
            pltpu.SemaphoreType.DMA,
        ],
    )
    def gmf(user_hbm, item_hbm, utab_hbm, itab_hbm, out_hbm,
            uidx_v, iidx_v, urows_v, irows_v, sem_u, sem_i):
        wid = lax.axis_index("s") * nc + lax.axis_index("c")
        pltpu.sync_copy(user_hbm.at[wid], uidx_v)
        pltpu.sync_copy(item_hbm.at[wid], iidx_v)
        copies = []
        for j in range(n_chunks):
            copies.append(
                pltpu.async_copy(utab_hbm.at[uidx_v.at[j]], urows_v.at[j], sem_u))
            copies.append(
                pltpu.async_copy(itab_hbm.at[iidx_v.at[j]], irows_v.at[j], sem_i))
        for cp in copies:
            cp.wait()

        def body(r, _):
            for j in range(n_chunks):
                urows_v[j, r] = urows_v[j, r] * irows_v[j, r]
            return 0

        lax.fori_loop(0, CHUNK, body, 0)
        pltpu.sync_copy(urows_v, out_hbm.at[wid])

    out = gmf(user2, item2, user_table, item_table)
    return out.reshape(BATCH, EMBED_DIM)


def kernel(user, item, user_table, item_table):
    return _gmf(user, item, user_table, item_table)
